# baseline (speedup 1.0000x reference)
import functools

import jax
import jax.numpy as jnp
from jax import lax
from jax.experimental import pallas as pl
from jax.experimental.pallas import tpu as pltpu
from jax.experimental.pallas import tpu_sc as plsc

NC = 2
NS = 16
NW = NC * NS
IW = 128
K = 4
SH = 256


def _emb_call(b, h, d):
    n = b * h
    n_per_w = n // NW
    s_per_w = b // NW
    c = K * IW
    n_groups = n_per_w // c
    assert n_groups % 2 == 0 and s_per_w == 2 * SH
    rep_per_h = (SH * h) // 16
    mesh = plsc.VectorSubcoreMesh(
        core_axis_name="c", subcore_axis_name="s",
        num_cores=NC, num_subcores=NS)

    @functools.partial(
        pl.kernel,
        out_type=jax.ShapeDtypeStruct((n, d), jnp.float32),
        mesh=mesh,
        scratch_types=[
            pltpu.VMEM((SH, 128), jnp.int32),
            pltpu.VMEM((n_per_w // IW, IW), jnp.int32),
            pltpu.VMEM((2, c, d), jnp.float32),
            pltpu.SemaphoreType.DMA,
            pltpu.SemaphoreType.DMA,
        ],
        compiler_params=pltpu.CompilerParams(
            use_tc_tiling_on_sc=False, needs_layout_passes=False),
    )
    def emb(ids_hbm, table_hbm, out_hbm, stage_v, idx_v, rows_v, sem0, sem1):
        wid = lax.axis_index("s") * NC + lax.axis_index("c")
        sent0 = wid * s_per_w
        base = wid * n_per_w
        lanes = lax.iota(jnp.int32, 16)

        for half in range(2):
            pltpu.sync_copy(
                ids_hbm.at[pl.ds(sent0 + half * SH, SH)], stage_v)
            f0 = half * SH * h

            def repack(i, carry):
                rel = lanes + lax.broadcast_in_dim(i * 16, (16,), ())
                hv = lax.broadcast_in_dim(jnp.int32(h), (16,), ())
                vals = plsc.load_gather(
                    stage_v, [lax.div(rel, hv), lax.rem(rel, hv)])
                f = f0 + i * 16
                idx_v[f // IW, pl.ds(f % IW, 16)] = vals
                return carry

            lax.fori_loop(0, rep_per_h, repack, 0)

        bufs = (rows_v.at[0], rows_v.at[1])
        sems = (sem0, sem1)

        def fire(g, bf):
            for j in range(K):
                pltpu.async_copy(
                    table_hbm.at[idx_v.at[g * K + j]],
                    bufs[bf].at[pl.ds(j * IW, IW)], sems[bf])

        def drain(g, bf):
            for j in range(K):
                pltpu.make_async_copy(
                    table_hbm.at[idx_v.at[g * K + j]],
                    bufs[bf].at[pl.ds(j * IW, IW)], sems[bf]).wait()
            pltpu.sync_copy(bufs[bf], out_hbm.at[pl.ds(base + g * c, c)])

        fire(0, 0)

        def step(i, carry):
            g = 2 * i
            fire(g + 1, 1)
            drain(g, 0)
            fire(g + 2, 0)
            drain(g + 1, 1)
            return carry

        lax.fori_loop(0, n_groups // 2 - 1, step, 0)
        g = n_groups - 2
        fire(g + 1, 1)
        drain(g, 0)
        drain(g + 1, 1)

    return emb


def kernel(token_ids, p_emb):
    b, h = token_ids.shape
    v, d = p_emb.shape
    ids128 = jnp.pad(token_ids.astype(jnp.int32), ((0, 0), (0, 128 - h)))
    out = _emb_call(b, h, d)(ids128, p_emb)
    return out.reshape(b, h, d)
